# baseline (speedup 1.0000x reference)
import functools

import jax
import jax.numpy as jnp
from jax import lax
from jax.experimental import pallas as pl
from jax.experimental.pallas import tpu as pltpu
from jax.experimental.pallas import tpu_sc as plsc

NUM_RELATIONS = 1000
FUSE_DIM = 64
BATCH = 16384

_NC = 2
_NS = 16
_NW = _NC * _NS
_CHUNK = BATCH // _NW
_SUB = 256
_NSUB = _CHUNK // _SUB
_PROWS = _SUB * FUSE_DIM // 128


def _sc_kernel(h_hbm, idx_hbm, t_hbm, rel_hbm, out_hbm,
               idx_v, h_v, t_v, r_v, out_v, sem_h, sem_t, sem_r):
    wid = lax.axis_index("s") * _NC + lax.axis_index("c")
    base = wid * _CHUNK

    iota = lax.iota(jnp.int32, 16)

    for sub in range(_NSUB):
        sbase = base + sub * _SUB
        pbase = pl.multiple_of(sbase * FUSE_DIM // 128, 8)
        pltpu.sync_copy(idx_hbm.at[pl.ds(sbase, _SUB)], idx_v)
        cp_r = pltpu.async_copy(rel_hbm.at[idx_v], r_v, sem_r)
        cp_h = pltpu.async_copy(h_hbm.at[pl.ds(pbase, _PROWS), :], h_v, sem_h)
        cp_t = pltpu.async_copy(t_hbm.at[pl.ds(pbase, _PROWS), :], t_v, sem_t)
        cp_h.wait()
        cp_t.wait()
        cp_r.wait()

        def group_body(g, carry):
            rows = iota + g * 16
            addr0 = rows * FUSE_DIM

            def d_body(d, acc):
                addr = addr0 + d
                prow = lax.shift_right_logical(addr, 7)
                pcol = lax.bitwise_and(addr, 127)
                cols = jnp.full((16,), d, dtype=jnp.int32)
                hv = plsc.load_gather(h_v, [prow, pcol])
                tv = plsc.load_gather(t_v, [prow, pcol])
                rv = plsc.load_gather(r_v, [rows, cols])
                return acc + hv * rv * tv

            acc = lax.fori_loop(0, FUSE_DIM, d_body,
                                jnp.zeros((16,), jnp.float32), unroll=8)
            out_v[pl.ds(sub * _SUB + g * 16, 16)] = acc
            return carry

        lax.fori_loop(0, _SUB // 16, group_body, 0)

    pltpu.sync_copy(out_v, out_hbm.at[pl.ds(base, _CHUNK)])


@jax.jit
def kernel(h_emb, r_idx, t_emb, rel_emb):
    h2 = h_emb.reshape(BATCH * FUSE_DIM // 128, 128)
    t2 = t_emb.reshape(BATCH * FUSE_DIM // 128, 128)
    relp = jnp.pad(rel_emb, ((0, 0), (0, 128 - FUSE_DIM)))
    mesh = plsc.VectorSubcoreMesh(core_axis_name="c", subcore_axis_name="s")
    run = functools.partial(
        pl.kernel,
        mesh=mesh,
        compiler_params=pltpu.CompilerParams(needs_layout_passes=False),
        out_type=jax.ShapeDtypeStruct((BATCH,), jnp.float32),
        scratch_types=[
            pltpu.VMEM((_SUB,), jnp.int32),
            pltpu.VMEM((_PROWS, 128), jnp.float32),
            pltpu.VMEM((_PROWS, 128), jnp.float32),
            pltpu.VMEM((_SUB, 128), jnp.float32),
            pltpu.VMEM((_CHUNK,), jnp.float32),
            pltpu.SemaphoreType.DMA,
            pltpu.SemaphoreType.DMA,
            pltpu.SemaphoreType.DMA,
        ],
    )(_sc_kernel)
    return run(h2, r_idx.astype(jnp.int32), t2, relp)


# speedup vs baseline: 1.8073x; 1.8073x over previous
import functools

import jax
import jax.numpy as jnp
from jax import lax
from jax.experimental import pallas as pl
from jax.experimental.pallas import tpu as pltpu
from jax.experimental.pallas import tpu_sc as plsc

NUM_RELATIONS = 1000
FUSE_DIM = 64
BATCH = 16384

_NC = 2
_NS = 16
_NW = _NC * _NS
_CHUNK = BATCH // _NW
_GROUPS = _CHUNK // 16


def _sc_kernel(h_hbm, idx_hbm, t_hbm, rel_hbm, out_hbm,
               idx_v, h_v, t_v, r_v, out_v, sem_h, sem_t, sem_r):
    wid = lax.axis_index("s") * _NC + lax.axis_index("c")
    base = wid * _CHUNK

    pltpu.sync_copy(idx_hbm.at[pl.ds(base, _CHUNK)], idx_v)
    cp_r = pltpu.async_copy(rel_hbm.at[idx_v], r_v, sem_r)
    cp_h = pltpu.async_copy(h_hbm.at[pl.ds(base, _CHUNK), :], h_v, sem_h)
    cp_t = pltpu.async_copy(t_hbm.at[pl.ds(base, _CHUNK), :], t_v, sem_t)
    cp_h.wait()
    cp_t.wait()
    cp_r.wait()

    iota = lax.iota(jnp.int32, 16)

    def group_body(g, carry):
        rows = iota + g * 16

        def d_body(dd, acc):
            cols = lax.bitwise_and(iota + dd, FUSE_DIM - 1)
            hv = plsc.load_gather(h_v, [rows, cols])
            rv = plsc.load_gather(r_v, [rows, cols])
            tv = plsc.load_gather(t_v, [rows, cols])
            return acc + hv * rv * tv

        acc = lax.fori_loop(0, FUSE_DIM, d_body, jnp.zeros((16,), jnp.float32),
                            unroll=8)
        out_v[pl.ds(g * 16, 16)] = acc
        return carry

    lax.fori_loop(0, _GROUPS, group_body, 0)
    pltpu.sync_copy(out_v, out_hbm.at[pl.ds(base, _CHUNK)])


@jax.jit
def kernel(h_emb, r_idx, t_emb, rel_emb):
    mesh = plsc.VectorSubcoreMesh(core_axis_name="c", subcore_axis_name="s")
    run = functools.partial(
        pl.kernel,
        mesh=mesh,
        compiler_params=pltpu.CompilerParams(
            needs_layout_passes=False, use_tc_tiling_on_sc=False),
        out_type=jax.ShapeDtypeStruct((BATCH,), jnp.float32),
        scratch_types=[
            pltpu.VMEM((_CHUNK,), jnp.int32),
            pltpu.VMEM((_CHUNK, FUSE_DIM), jnp.float32),
            pltpu.VMEM((_CHUNK, FUSE_DIM), jnp.float32),
            pltpu.VMEM((_CHUNK, FUSE_DIM), jnp.float32),
            pltpu.VMEM((_CHUNK,), jnp.float32),
            pltpu.SemaphoreType.DMA,
            pltpu.SemaphoreType.DMA,
            pltpu.SemaphoreType.DMA,
        ],
    )(_sc_kernel)
    return run(h_emb, r_idx.astype(jnp.int32), t_emb, rel_emb)


# speedup vs baseline: 1.8073x; 1.0000x over previous
import functools

import jax
import jax.numpy as jnp
from jax import lax
from jax.experimental import pallas as pl
from jax.experimental.pallas import tpu as pltpu
from jax.experimental.pallas import tpu_sc as plsc

NUM_RELATIONS = 1000
FUSE_DIM = 64
BATCH = 16384

_NC = 2
_NS = 16
_NW = _NC * _NS
_CHUNK = BATCH // _NW
_GROUPS = _CHUNK // 16


def _sc_kernel(h_hbm, idx_hbm, t_hbm, rel_hbm, out_hbm,
               idx_v, h_v, t_v, r_v, out_v, sem_h, sem_t, sem_r):
    wid = lax.axis_index("s") * _NC + lax.axis_index("c")
    base = wid * _CHUNK

    pltpu.sync_copy(idx_hbm.at[pl.ds(base, _CHUNK)], idx_v)
    cp_r = pltpu.async_copy(rel_hbm.at[idx_v], r_v, sem_r)
    cp_h = pltpu.async_copy(h_hbm.at[pl.ds(base, _CHUNK), :], h_v, sem_h)
    cp_t = pltpu.async_copy(t_hbm.at[pl.ds(base, _CHUNK), :], t_v, sem_t)
    cp_h.wait()
    cp_t.wait()
    cp_r.wait()

    iota = lax.iota(jnp.int32, 16)

    def group_body(g, carry):
        rows = iota + g * 16

        def d_body(dd, acc):
            cols = lax.bitwise_and(iota + dd, FUSE_DIM - 1)
            hv = plsc.load_gather(h_v, [rows, cols])
            rv = plsc.load_gather(r_v, [rows, cols])
            tv = plsc.load_gather(t_v, [rows, cols])
            return acc + hv * rv * tv

        acc = lax.fori_loop(0, FUSE_DIM, d_body, jnp.zeros((16,), jnp.float32),
                            unroll=8)
        out_v[pl.ds(g * 16, 16)] = acc
        return carry

    lax.fori_loop(0, _GROUPS, group_body, 0)
    pltpu.sync_copy(out_v, out_hbm.at[pl.ds(base, _CHUNK)])


@jax.jit
def kernel(h_emb, r_idx, t_emb, rel_emb):
    mesh = plsc.VectorSubcoreMesh(core_axis_name="c", subcore_axis_name="s")
    run = functools.partial(
        pl.kernel,
        mesh=mesh,
        compiler_params=pltpu.CompilerParams(
            needs_layout_passes=False, use_tc_tiling_on_sc=False,
            skip_device_barrier=True),
        out_type=jax.ShapeDtypeStruct((BATCH,), jnp.float32),
        scratch_types=[
            pltpu.VMEM((_CHUNK,), jnp.int32),
            pltpu.VMEM((_CHUNK, FUSE_DIM), jnp.float32),
            pltpu.VMEM((_CHUNK, FUSE_DIM), jnp.float32),
            pltpu.VMEM((_CHUNK, FUSE_DIM), jnp.float32),
            pltpu.VMEM((_CHUNK,), jnp.float32),
            pltpu.SemaphoreType.DMA,
            pltpu.SemaphoreType.DMA,
            pltpu.SemaphoreType.DMA,
        ],
    )(_sc_kernel)
    return run(h_emb, r_idx.astype(jnp.int32), t_emb, rel_emb)
